# baseline (speedup 1.0000x reference)
import functools
import math

import jax
import jax.numpy as jnp
from jax import lax
from jax.experimental import pallas as pl
from jax.experimental.pallas import tpu as pltpu
from jax.experimental.pallas import tpu_sc as plsc

NROW, NCOL = 4096, 200
V = 1000000
D = 64
LANES = 16
NC, NS = 2, 16
NW = NC * NS
SCALE = math.sqrt(D)

CR = 192
NCH_FULL = V // CR
REM_ROWS = V - NCH_FULL * CR
KMAX = -(-NCH_FULL // NW)

RPW = NROW // NW
G = RPW
TAIL = NCOL - 128

_mesh = plsc.VectorSubcoreMesh(core_axis_name="c", subcore_axis_name="s")


@functools.partial(
    pl.kernel,
    mesh=_mesh,
    out_type=jax.ShapeDtypeStruct((V, 128), jnp.float32),
    scratch_types=[
        pltpu.VMEM((2, CR, D), jnp.float32),
        pltpu.VMEM((2, CR, 128), jnp.float32),
        pltpu.SemaphoreType.DMA,
        pltpu.SemaphoreType.DMA,
        pltpu.SemaphoreType.DMA,
        pltpu.SemaphoreType.DMA,
    ],
    compiler_params=pltpu.CompilerParams(use_tc_tiling_on_sc=True),
)
def _pack(table_hbm, packed_hbm, bufr, bufw, r0, r1, w0, w1):
    wid = lax.axis_index("s") * NC + lax.axis_index("c")
    rsem = (r0, r1)
    wsem = (w0, w1)

    def chunk_off(k):
        return pl.multiple_of((wid + k * NW) * CR, 8)

    def read(k, b):
        pltpu.async_copy(table_hbm.at[pl.ds(chunk_off(k), CR)], bufr.at[b],
                         rsem[b])

    def wait_read(b):
        pltpu.make_async_copy(
            table_hbm.at[pl.ds(0, CR)], bufr.at[b], rsem[b]
        ).wait()

    def write(k, b):
        pltpu.async_copy(
            bufw.at[b], packed_hbm.at[pl.ds(chunk_off(k), CR)], wsem[b]
        )

    def wait_write(b):
        pltpu.make_async_copy(
            bufw.at[b], packed_hbm.at[pl.ds(0, CR)], wsem[b]
        ).wait()

    nk = jnp.where(wid < NCH_FULL - (KMAX - 1) * NW, KMAX, KMAX - 1)
    read(0, 0)

    def body(k, carry):
        for b in (0, 1):
            kc = 2 * k + b

            @pl.when(kc < nk)
            def _():
                @pl.when(kc + 1 < nk)
                def _():
                    read(kc + 1, 1 - b)

                wait_read(b)

                @pl.when(kc >= 2)
                def _():
                    wait_write(b)

                @plsc.parallel_loop(0, CR, 1, unroll=8)
                def _(c):
                    for kk in range(D // LANES):
                        sl = pl.ds(kk * LANES, LANES)
                        bufw[b, c, sl] = bufr[b, c, sl]

                write(kc, b)
        return carry

    lax.fori_loop(0, (KMAX + 1) // 2, body, 0)

    @pl.when(nk >= 2)
    def _():
        wait_write(0)
        wait_write(1)

    @pl.when(nk == 1)
    def _():
        wait_write(0)

    @pl.when(wid == 0)
    def _():
        off = pl.multiple_of(NCH_FULL * CR, 8)
        pltpu.sync_copy(
            table_hbm.at[pl.ds(off, REM_ROWS)], bufr.at[0, pl.ds(0, REM_ROWS)]
        )

        @plsc.parallel_loop(0, REM_ROWS, 1, unroll=8)
        def _(c):
            for kk in range(D // LANES):
                sl = pl.ds(kk * LANES, LANES)
                bufw[0, c, sl] = bufr[0, c, sl]

        pltpu.sync_copy(
            bufw.at[0, pl.ds(0, REM_ROWS)], packed_hbm.at[pl.ds(off, REM_ROWS)]
        )


@functools.partial(
    pl.kernel,
    mesh=_mesh,
    out_type=jax.ShapeDtypeStruct((NROW, NCOL, D), jnp.float32),
    scratch_types=[
        pltpu.VMEM((2, 8, NCOL), jnp.int32),
        pltpu.VMEM((2, 8, 128), jnp.int32),
        pltpu.VMEM((2, NCOL, 128), jnp.float32),
        pltpu.VMEM((2, 1, NCOL, D), jnp.float32),
        pltpu.SemaphoreType.DMA,
        pltpu.SemaphoreType.DMA,
        pltpu.SemaphoreType.DMA,
        pltpu.SemaphoreType.DMA,
    ],
    compiler_params=pltpu.CompilerParams(use_tc_tiling_on_sc=True),
)
def _lookup(x_hbm, packed_hbm, out_hbm, idxf, idxb, rows, stg,
            g0, g1, o0, o1):
    wid = lax.axis_index("s") * NC + lax.axis_index("c")
    row_base = wid * RPW
    gsem = (g0, g1)
    osem = (o0, o1)

    def load_group(gc):
        grp = gc // 8
        gb = grp % 2
        r0 = pl.multiple_of(row_base + grp * 8, 8)
        pltpu.sync_copy(x_hbm.at[pl.ds(r0, 8)], idxf.at[gb])
        for r in range(8):
            for k in range(8):
                idxb[gb, r, pl.ds(k * LANES, LANES)] = (
                    idxf[gb, r, pl.ds(NCOL - 128 + k * LANES, LANES)]
                )

    def fire(gc, b):
        grp_b = (gc // 8) % 2
        r = gc % 8
        pltpu.async_copy(
            packed_hbm.at[idxf.at[grp_b, r, pl.ds(0, 128)]],
            rows.at[b, pl.ds(0, 128)],
            gsem[b],
        )
        pltpu.async_copy(
            packed_hbm.at[idxb.at[grp_b, r]],
            rows.at[b, pl.ds(NCOL - 128, 128)],
            gsem[b],
        )

    def wait_gathers(b):
        pltpu.make_async_copy(
            packed_hbm.at[idxf.at[0, 0, pl.ds(0, 128)]],
            rows.at[b, pl.ds(0, 128)],
            gsem[b],
        ).wait()
        pltpu.make_async_copy(
            packed_hbm.at[idxb.at[0, 0]],
            rows.at[b, pl.ds(NCOL - 128, 128)],
            gsem[b],
        ).wait()

    def wait_writeback(b):
        pltpu.make_async_copy(
            stg.at[b], out_hbm.at[pl.ds(0, 1)], osem[b]
        ).wait()

    load_group(0)
    fire(0, 0)

    def pair_body(k, carry):
        for b in (0, 1):
            gc = 2 * k + b
            nxt = gc + 1

            @pl.when(nxt < G)
            def _():
                @pl.when(nxt % 8 == 0)
                def _():
                    load_group(nxt)

                @pl.when(nxt >= 2)
                def _():
                    wait_writeback(1 - b)

                fire(nxt, 1 - b)

            wait_gathers(b)

            @plsc.parallel_loop(0, NCOL, 1, unroll=8)
            def _(c):
                for kk in range(D // LANES):
                    sl = pl.ds(kk * LANES, LANES)
                    stg[b, 0, c, sl] = rows[b, c, sl] * SCALE

            pltpu.async_copy(
                stg.at[b],
                out_hbm.at[pl.ds(row_base + gc, 1)],
                osem[b],
            )
        return carry

    lax.fori_loop(0, G // 2, pair_body, 0)
    wait_writeback(0)
    wait_writeback(1)


def kernel(x, table):
    packed = _pack(table)
    return _lookup(x.astype(jnp.int32), packed)
